# baseline (speedup 1.0000x reference)
import functools

import jax
import jax.numpy as jnp
from jax import lax
from jax.experimental import pallas as pl
from jax.experimental.pallas import tpu as pltpu
from jax.experimental.pallas import tpu_sc as plsc

N = 10000
E = 160000
D = 256
HALF = 128
NC = 2
NS = 16
RPT = N // NS
NCH = E // 128
K = 4

_MESH = dict(core_axis_name="c", subcore_axis_name="s", num_cores=NC,
             num_subcores=NS)
_SC_PARAMS = pltpu.CompilerParams(use_tc_tiling_on_sc=False)


def _deg_sc(dst2d, ones16, zeros16):
    mesh = plsc.VectorSubcoreMesh(**_MESH)

    @functools.partial(
        pl.kernel, mesh=mesh,
        out_type=jax.ShapeDtypeStruct((NC, N, 16), jnp.float32),
        compiler_params=_SC_PARAMS,
        scratch_types=[
            pltpu.VMEM((40, 128), jnp.int32),
            pltpu.VMEM((128, 16), jnp.float32),
            pltpu.SemaphoreType.DMA,
            pltpu.VMEM_SHARED((N, 16), jnp.float32),
        ],
    )
    def k(dst_hbm, ones_hbm, zeros_hbm, out_hbm, dst_v, ones_v, sem, acc_sh):
        c = lax.axis_index("c")
        s = lax.axis_index("s")
        w = s * NC + c
        off = w * 39 + jnp.minimum(w, 2)
        nc = 39 + (w < 2).astype(jnp.int32)
        pltpu.sync_copy(dst_hbm.at[pl.ds(off, 39)], dst_v.at[pl.ds(0, 39)])

        @pl.when(w < 2)
        def _():
            pltpu.sync_copy(dst_hbm.at[pl.ds(off + 39, 1)],
                            dst_v.at[pl.ds(39, 1)])

        pltpu.sync_copy(ones_hbm, ones_v)
        pltpu.sync_copy(zeros_hbm, acc_sh.at[pl.ds(s * RPT, RPT)])
        plsc.subcore_barrier()

        def fire(j, carry):
            @pl.when(j < nc)
            def _():
                pltpu.async_copy(ones_v, acc_sh.at[dst_v.at[j]], sem, add=True)
            return carry

        def drain(j, carry):
            @pl.when(j < nc)
            def _():
                pltpu.make_async_copy(ones_v, acc_sh.at[dst_v.at[0]],
                                      sem).wait()
            return carry

        lax.fori_loop(0, 40, fire, 0)
        lax.fori_loop(0, 40, drain, 0)
        plsc.subcore_barrier()
        pltpu.sync_copy(acc_sh.at[pl.ds(s * RPT, RPT)],
                        out_hbm.at[c, pl.ds(s * RPT, RPT)])

    return k(dst2d, ones16, zeros16)


def _agg1_sc(H, src2d, dst2d, zeros128):
    mesh = plsc.VectorSubcoreMesh(**_MESH)
    CH = 80
    HALVES = (63, 62)

    @functools.partial(
        pl.kernel, mesh=mesh,
        out_type=jax.ShapeDtypeStruct((NC, N, HALF), jnp.float32),
        compiler_params=_SC_PARAMS,
        scratch_types=[
            pltpu.VMEM((63, CH), jnp.int32),
            pltpu.VMEM((63, CH), jnp.int32),
            pltpu.VMEM((K, CH, HALF), jnp.float32),
            [pltpu.SemaphoreType.DMA] * K,
            [pltpu.SemaphoreType.DMA] * K,
            pltpu.VMEM_SHARED((N, HALF), jnp.float32),
        ],
    )
    def k(h_hbm, src_hbm, dst_hbm, zeros_hbm, out_hbm,
          src_v, dst_v, buf, gsem, ssem, acc_sh):
        c = lax.axis_index("c")
        s = lax.axis_index("s")
        tab = h_hbm.at[c]
        pltpu.sync_copy(zeros_hbm, acc_sh.at[pl.ds(s * RPT, RPT)])
        plsc.subcore_barrier()

        for half in range(2):
            hnc = HALVES[half]
            base = s * 125 + half * HALVES[0]
            pltpu.sync_copy(src_hbm.at[pl.ds(base, hnc)],
                            src_v.at[pl.ds(0, hnc)])
            pltpu.sync_copy(dst_hbm.at[pl.ds(base, hnc)],
                            dst_v.at[pl.ds(0, hnc)])
            for b in range(K):
                pltpu.async_copy(tab.at[src_v.at[b]], buf.at[b], gsem[b])

            def rnd(g, carry, hnc=hnc):
                for b in range(K):
                    j = g * K + b

                    @pl.when(j < hnc)
                    def _(b=b, j=j):
                        pltpu.make_async_copy(tab.at[src_v.at[j]], buf.at[b],
                                              gsem[b]).wait()
                        pltpu.async_copy(buf.at[b], acc_sh.at[dst_v.at[j]],
                                         ssem[b], add=True)
                for b in range(K):
                    j2 = (g + 1) * K + b

                    @pl.when(j2 < hnc)
                    def _(b=b, j2=j2):
                        pltpu.make_async_copy(buf.at[b],
                                              acc_sh.at[dst_v.at[0]],
                                              ssem[b]).wait()
                        pltpu.async_copy(tab.at[src_v.at[j2]], buf.at[b],
                                         gsem[b])
                return carry

            lax.fori_loop(0, (hnc + K - 1) // K, rnd, 0)
            for b in range(K):
                pltpu.make_async_copy(buf.at[b], acc_sh.at[dst_v.at[0]],
                                      ssem[b]).wait()
        plsc.subcore_barrier()
        pltpu.sync_copy(acc_sh.at[pl.ds(s * RPT, RPT)],
                        out_hbm.at[c, pl.ds(s * RPT, RPT)])

    return k(H, src2d, dst2d, zeros128)


def _agg2_sc(g, src2d, dst2d, zeros16):
    mesh = plsc.VectorSubcoreMesh(**_MESH)

    @functools.partial(
        pl.kernel, mesh=mesh,
        out_type=jax.ShapeDtypeStruct((NC, N, 16), jnp.float32),
        compiler_params=_SC_PARAMS,
        scratch_types=[
            pltpu.VMEM((40, 128), jnp.int32),
            pltpu.VMEM((40, 128), jnp.int32),
            pltpu.VMEM((40, 128, 16), jnp.float32),
            pltpu.SemaphoreType.DMA,
            pltpu.SemaphoreType.DMA,
            pltpu.VMEM_SHARED((N, 16), jnp.float32),
        ],
    )
    def k(g_hbm, src_hbm, dst_hbm, zeros_hbm, out_hbm,
          src_v, dst_v, buf, gsem, ssem, acc_sh):
        c = lax.axis_index("c")
        s = lax.axis_index("s")
        w = s * NC + c
        off = w * 39 + jnp.minimum(w, 2)
        nc = 39 + (w < 2).astype(jnp.int32)
        pltpu.sync_copy(src_hbm.at[pl.ds(off, 39)], src_v.at[pl.ds(0, 39)])
        pltpu.sync_copy(dst_hbm.at[pl.ds(off, 39)], dst_v.at[pl.ds(0, 39)])

        @pl.when(w < 2)
        def _():
            pltpu.sync_copy(src_hbm.at[pl.ds(off + 39, 1)],
                            src_v.at[pl.ds(39, 1)])
            pltpu.sync_copy(dst_hbm.at[pl.ds(off + 39, 1)],
                            dst_v.at[pl.ds(39, 1)])

        pltpu.sync_copy(zeros_hbm, acc_sh.at[pl.ds(s * RPT, RPT)])
        plsc.subcore_barrier()

        def gfire(j, carry):
            @pl.when(j < nc)
            def _():
                pltpu.async_copy(g_hbm.at[src_v.at[j]], buf.at[j], gsem)
            return carry

        def gdrain(j, carry):
            @pl.when(j < nc)
            def _():
                pltpu.make_async_copy(g_hbm.at[src_v.at[0]], buf.at[0],
                                      gsem).wait()
            return carry

        def sfire(j, carry):
            @pl.when(j < nc)
            def _():
                pltpu.async_copy(buf.at[j], acc_sh.at[dst_v.at[j]], ssem,
                                 add=True)
            return carry

        def sdrain(j, carry):
            @pl.when(j < nc)
            def _():
                pltpu.make_async_copy(buf.at[0], acc_sh.at[dst_v.at[0]],
                                      ssem).wait()
            return carry

        lax.fori_loop(0, 40, gfire, 0)
        lax.fori_loop(0, 40, gdrain, 0)
        lax.fori_loop(0, 40, sfire, 0)
        lax.fori_loop(0, 40, sdrain, 0)
        plsc.subcore_barrier()
        pltpu.sync_copy(acc_sh.at[pl.ds(s * RPT, RPT)],
                        out_hbm.at[c, pl.ds(s * RPT, RPT)])

    return k(g, src2d, dst2d, zeros16)


_BLK = 2000
_PBLK = _BLK // 16


def _split_edges(edge_index):
    def body(e_ref, s_ref, d_ref):
        s_ref[...] = e_ref[0]
        d_ref[...] = e_ref[1]

    return pl.pallas_call(
        body,
        grid=(1,),
        in_specs=[pl.BlockSpec((2, E), lambda i: (0, 0))],
        out_specs=[pl.BlockSpec((E,), lambda i: (0,)),
                   pl.BlockSpec((E,), lambda i: (0,))],
        out_shape=[jax.ShapeDtypeStruct((E,), jnp.int32),
                   jax.ShapeDtypeStruct((E,), jnp.int32)],
    )(edge_index)


def _tc1(x, W1, degp):
    def body(x_ref, w_ref, deg_ref, h_ref, dinv_ref):
        deg = deg_ref[0, :, 0:1] + deg_ref[1, :, 0:1] + 1.0
        dinv = lax.rsqrt(deg)
        h = jnp.dot(x_ref[...], w_ref[...],
                    preferred_element_type=jnp.float32,
                    precision=lax.Precision.HIGHEST)
        h_ref[0] = h[:, 0:HALF] * dinv
        h_ref[1] = h[:, HALF:D] * dinv
        dinv_ref[...] = dinv

    return pl.pallas_call(
        body,
        grid=(N // _BLK,),
        in_specs=[
            pl.BlockSpec((_BLK, D), lambda i: (i, 0)),
            pl.BlockSpec((D, D), lambda i: (0, 0)),
            pl.BlockSpec((NC, _BLK, 16), lambda i: (0, i, 0)),
        ],
        out_specs=[
            pl.BlockSpec((NC, _BLK, HALF), lambda i: (0, i, 0)),
            pl.BlockSpec((_BLK, 1), lambda i: (i, 0)),
        ],
        out_shape=[
            jax.ShapeDtypeStruct((NC, N, HALF), jnp.float32),
            jax.ShapeDtypeStruct((N, 1), jnp.float32),
        ],
    )(x, W1, degp)


def _tc2(acc1, H, dinv, b1r, W2p):
    def body(a_ref, h_ref, dinv_ref, b_ref, w_ref, g_ref):
        dinv = dinv_ref[...]
        t0 = jnp.maximum(dinv * (a_ref[0] + h_ref[0]) + b_ref[0:1, 0:HALF], 0.0)
        t1 = jnp.maximum(dinv * (a_ref[1] + h_ref[1]) + b_ref[0:1, HALF:D], 0.0)
        g = (jnp.dot(t0, w_ref[0:HALF, :], preferred_element_type=jnp.float32,
                     precision=lax.Precision.HIGHEST)
             + jnp.dot(t1, w_ref[HALF:D, :], preferred_element_type=jnp.float32,
                       precision=lax.Precision.HIGHEST))
        g_ref[...] = g * dinv

    return pl.pallas_call(
        body,
        grid=(N // _BLK,),
        in_specs=[
            pl.BlockSpec((NC, _BLK, HALF), lambda i: (0, i, 0)),
            pl.BlockSpec((NC, _BLK, HALF), lambda i: (0, i, 0)),
            pl.BlockSpec((_BLK, 1), lambda i: (i, 0)),
            pl.BlockSpec((1, D), lambda i: (0, 0)),
            pl.BlockSpec((D, 16), lambda i: (0, 0)),
        ],
        out_specs=pl.BlockSpec((_BLK, 16), lambda i: (i, 0)),
        out_shape=jax.ShapeDtypeStruct((N, 16), jnp.float32),
    )(acc1, H, dinv, b1r, W2p)


def _tc3(acc2, g, dinv, b2p):
    def body(a_ref, g_ref, dinv_ref, b_ref, o_ref):
        o = dinv_ref[...] * (a_ref[0] + a_ref[1] + g_ref[...]) + b_ref[...]
        o_ref[...] = o[:, 0:3]

    return pl.pallas_call(
        body,
        grid=(1,),
        in_specs=[
            pl.BlockSpec((NC, N, 16), lambda i: (0, 0, 0)),
            pl.BlockSpec((N, 16), lambda i: (0, 0)),
            pl.BlockSpec((N, 1), lambda i: (0, 0)),
            pl.BlockSpec((1, 16), lambda i: (0, 0)),
        ],
        out_specs=pl.BlockSpec((N, 3), lambda i: (0, 0)),
        out_shape=jax.ShapeDtypeStruct((N, 3), jnp.float32),
    )(acc2, g, dinv, b2p)


def kernel(x, edge_index, W1, b1, W2, b2):
    src1d, dst1d = _split_edges(edge_index.astype(jnp.int32))
    src2d = src1d.reshape(NCH, 128)
    dst2d = dst1d.reshape(NCH, 128)
    src40 = src1d.reshape(E // 80, 80)
    dst40 = dst1d.reshape(E // 80, 80)

    zeros16 = jnp.zeros((RPT, 16), jnp.float32)
    zeros128 = jnp.zeros((RPT, HALF), jnp.float32)
    ones16 = jnp.ones((128, 16), jnp.float32)
    W2p = jnp.pad(W2, ((0, 0), (0, 16 - W2.shape[1])))
    b1r = b1.reshape(1, D)
    b2p = jnp.pad(b2, (0, 16 - b2.shape[0])).reshape(1, 16)

    degp = _deg_sc(dst2d, ones16, zeros16)
    H, dinv = _tc1(x, W1, degp)
    acc1 = _agg1_sc(H, src40, dst40, zeros128)
    g = _tc2(acc1, H, dinv, b1r, W2p)
    acc2 = _agg2_sc(g, src2d, dst2d, zeros16)
    return _tc3(acc2, g, dinv, b2p)


# speedup vs baseline: 1.0202x; 1.0202x over previous
import functools

import jax
import jax.numpy as jnp
from jax import lax
from jax.experimental import pallas as pl
from jax.experimental.pallas import tpu as pltpu
from jax.experimental.pallas import tpu_sc as plsc

N = 10000
E = 160000
D = 256
HALF = 128
NC = 2
NS = 16
RPT = N // NS
NCH = E // 128
K = 8

_MESH = dict(core_axis_name="c", subcore_axis_name="s", num_cores=NC,
             num_subcores=NS)
_SC_PARAMS = pltpu.CompilerParams(use_tc_tiling_on_sc=False)


def _deg_sc(dst2d, ones16, zeros16):
    mesh = plsc.VectorSubcoreMesh(**_MESH)

    @functools.partial(
        pl.kernel, mesh=mesh,
        out_type=jax.ShapeDtypeStruct((NC, N, 16), jnp.float32),
        compiler_params=_SC_PARAMS,
        scratch_types=[
            pltpu.VMEM((40, 128), jnp.int32),
            pltpu.VMEM((128, 16), jnp.float32),
            pltpu.SemaphoreType.DMA,
            pltpu.VMEM_SHARED((N, 16), jnp.float32),
        ],
    )
    def k(dst_hbm, ones_hbm, zeros_hbm, out_hbm, dst_v, ones_v, sem, acc_sh):
        c = lax.axis_index("c")
        s = lax.axis_index("s")
        w = s * NC + c
        off = w * 39 + jnp.minimum(w, 2)
        nc = 39 + (w < 2).astype(jnp.int32)
        pltpu.sync_copy(dst_hbm.at[pl.ds(off, 39)], dst_v.at[pl.ds(0, 39)])

        @pl.when(w < 2)
        def _():
            pltpu.sync_copy(dst_hbm.at[pl.ds(off + 39, 1)],
                            dst_v.at[pl.ds(39, 1)])

        pltpu.sync_copy(ones_hbm, ones_v)
        pltpu.sync_copy(zeros_hbm, acc_sh.at[pl.ds(s * RPT, RPT)])
        plsc.subcore_barrier()

        def fire(j, carry):
            @pl.when(j < nc)
            def _():
                pltpu.async_copy(ones_v, acc_sh.at[dst_v.at[j]], sem, add=True)
            return carry

        def drain(j, carry):
            @pl.when(j < nc)
            def _():
                pltpu.make_async_copy(ones_v, acc_sh.at[dst_v.at[0]],
                                      sem).wait()
            return carry

        lax.fori_loop(0, 40, fire, 0)
        lax.fori_loop(0, 40, drain, 0)
        plsc.subcore_barrier()
        pltpu.sync_copy(acc_sh.at[pl.ds(s * RPT, RPT)],
                        out_hbm.at[c, pl.ds(s * RPT, RPT)])

    return k(dst2d, ones16, zeros16)


def _agg1_sc(H, src2d, dst2d, zeros128):
    mesh = plsc.VectorSubcoreMesh(**_MESH)
    CH = 40
    HALVES = (125, 125)

    @functools.partial(
        pl.kernel, mesh=mesh,
        out_type=jax.ShapeDtypeStruct((NC, N, HALF), jnp.float32),
        compiler_params=_SC_PARAMS,
        scratch_types=[
            pltpu.VMEM((125, CH), jnp.int32),
            pltpu.VMEM((125, CH), jnp.int32),
            pltpu.VMEM((K, CH, HALF), jnp.float32),
            [pltpu.SemaphoreType.DMA] * K,
            [pltpu.SemaphoreType.DMA] * K,
            pltpu.VMEM_SHARED((N, HALF), jnp.float32),
        ],
    )
    def k(h_hbm, src_hbm, dst_hbm, zeros_hbm, out_hbm,
          src_v, dst_v, buf, gsem, ssem, acc_sh):
        c = lax.axis_index("c")
        s = lax.axis_index("s")
        tab = h_hbm.at[c]
        pltpu.sync_copy(zeros_hbm, acc_sh.at[pl.ds(s * RPT, RPT)])
        plsc.subcore_barrier()

        for half in range(2):
            hnc = HALVES[half]
            base = s * 250 + half * HALVES[0]
            pltpu.sync_copy(src_hbm.at[pl.ds(base, hnc)],
                            src_v.at[pl.ds(0, hnc)])
            pltpu.sync_copy(dst_hbm.at[pl.ds(base, hnc)],
                            dst_v.at[pl.ds(0, hnc)])
            for b in range(K):
                pltpu.async_copy(tab.at[src_v.at[b]], buf.at[b], gsem[b])

            def rnd(g, carry, hnc=hnc):
                for b in range(K):
                    j = g * K + b

                    @pl.when(j < hnc)
                    def _(b=b, j=j):
                        pltpu.make_async_copy(tab.at[src_v.at[j]], buf.at[b],
                                              gsem[b]).wait()
                        pltpu.async_copy(buf.at[b], acc_sh.at[dst_v.at[j]],
                                         ssem[b], add=True)
                for b in range(K):
                    j2 = (g + 1) * K + b

                    @pl.when(j2 < hnc)
                    def _(b=b, j2=j2):
                        pltpu.make_async_copy(buf.at[b],
                                              acc_sh.at[dst_v.at[0]],
                                              ssem[b]).wait()
                        pltpu.async_copy(tab.at[src_v.at[j2]], buf.at[b],
                                         gsem[b])
                return carry

            lax.fori_loop(0, (hnc + K - 1) // K, rnd, 0)
            for b in range(K):
                pltpu.make_async_copy(buf.at[b], acc_sh.at[dst_v.at[0]],
                                      ssem[b]).wait()
        plsc.subcore_barrier()
        pltpu.sync_copy(acc_sh.at[pl.ds(s * RPT, RPT)],
                        out_hbm.at[c, pl.ds(s * RPT, RPT)])

    return k(H, src2d, dst2d, zeros128)


def _agg2_sc(g, src2d, dst2d, zeros16):
    mesh = plsc.VectorSubcoreMesh(**_MESH)

    @functools.partial(
        pl.kernel, mesh=mesh,
        out_type=jax.ShapeDtypeStruct((NC, N, 16), jnp.float32),
        compiler_params=_SC_PARAMS,
        scratch_types=[
            pltpu.VMEM((40, 128), jnp.int32),
            pltpu.VMEM((40, 128), jnp.int32),
            pltpu.VMEM((40, 128, 16), jnp.float32),
            pltpu.SemaphoreType.DMA,
            pltpu.SemaphoreType.DMA,
            pltpu.VMEM_SHARED((N, 16), jnp.float32),
        ],
    )
    def k(g_hbm, src_hbm, dst_hbm, zeros_hbm, out_hbm,
          src_v, dst_v, buf, gsem, ssem, acc_sh):
        c = lax.axis_index("c")
        s = lax.axis_index("s")
        w = s * NC + c
        off = w * 39 + jnp.minimum(w, 2)
        nc = 39 + (w < 2).astype(jnp.int32)
        pltpu.sync_copy(src_hbm.at[pl.ds(off, 39)], src_v.at[pl.ds(0, 39)])
        pltpu.sync_copy(dst_hbm.at[pl.ds(off, 39)], dst_v.at[pl.ds(0, 39)])

        @pl.when(w < 2)
        def _():
            pltpu.sync_copy(src_hbm.at[pl.ds(off + 39, 1)],
                            src_v.at[pl.ds(39, 1)])
            pltpu.sync_copy(dst_hbm.at[pl.ds(off + 39, 1)],
                            dst_v.at[pl.ds(39, 1)])

        pltpu.sync_copy(zeros_hbm, acc_sh.at[pl.ds(s * RPT, RPT)])
        plsc.subcore_barrier()

        def gfire(j, carry):
            @pl.when(j < nc)
            def _():
                pltpu.async_copy(g_hbm.at[src_v.at[j]], buf.at[j], gsem)
            return carry

        def gdrain(j, carry):
            @pl.when(j < nc)
            def _():
                pltpu.make_async_copy(g_hbm.at[src_v.at[0]], buf.at[0],
                                      gsem).wait()
            return carry

        def sfire(j, carry):
            @pl.when(j < nc)
            def _():
                pltpu.async_copy(buf.at[j], acc_sh.at[dst_v.at[j]], ssem,
                                 add=True)
            return carry

        def sdrain(j, carry):
            @pl.when(j < nc)
            def _():
                pltpu.make_async_copy(buf.at[0], acc_sh.at[dst_v.at[0]],
                                      ssem).wait()
            return carry

        lax.fori_loop(0, 40, gfire, 0)
        lax.fori_loop(0, 40, gdrain, 0)
        lax.fori_loop(0, 40, sfire, 0)
        lax.fori_loop(0, 40, sdrain, 0)
        plsc.subcore_barrier()
        pltpu.sync_copy(acc_sh.at[pl.ds(s * RPT, RPT)],
                        out_hbm.at[c, pl.ds(s * RPT, RPT)])

    return k(g, src2d, dst2d, zeros16)


_BLK = 2000
_PBLK = _BLK // 16


def _split_edges(edge_index):
    def body(e_ref, s_ref, d_ref):
        s_ref[...] = e_ref[0]
        d_ref[...] = e_ref[1]

    return pl.pallas_call(
        body,
        grid=(1,),
        in_specs=[pl.BlockSpec((2, E), lambda i: (0, 0))],
        out_specs=[pl.BlockSpec((E,), lambda i: (0,)),
                   pl.BlockSpec((E,), lambda i: (0,))],
        out_shape=[jax.ShapeDtypeStruct((E,), jnp.int32),
                   jax.ShapeDtypeStruct((E,), jnp.int32)],
    )(edge_index)


def _tc1(x, W1, degp):
    def body(x_ref, w_ref, deg_ref, h_ref, dinv_ref):
        deg = deg_ref[0, :, 0:1] + deg_ref[1, :, 0:1] + 1.0
        dinv = lax.rsqrt(deg)
        h = jnp.dot(x_ref[...], w_ref[...],
                    preferred_element_type=jnp.float32,
                    precision=lax.Precision.HIGHEST)
        h_ref[0] = h[:, 0:HALF] * dinv
        h_ref[1] = h[:, HALF:D] * dinv
        dinv_ref[...] = dinv

    return pl.pallas_call(
        body,
        grid=(N // _BLK,),
        in_specs=[
            pl.BlockSpec((_BLK, D), lambda i: (i, 0)),
            pl.BlockSpec((D, D), lambda i: (0, 0)),
            pl.BlockSpec((NC, _BLK, 16), lambda i: (0, i, 0)),
        ],
        out_specs=[
            pl.BlockSpec((NC, _BLK, HALF), lambda i: (0, i, 0)),
            pl.BlockSpec((_BLK, 1), lambda i: (i, 0)),
        ],
        out_shape=[
            jax.ShapeDtypeStruct((NC, N, HALF), jnp.float32),
            jax.ShapeDtypeStruct((N, 1), jnp.float32),
        ],
    )(x, W1, degp)


def _tc2(acc1, H, dinv, b1r, W2p):
    def body(a_ref, h_ref, dinv_ref, b_ref, w_ref, g_ref):
        dinv = dinv_ref[...]
        t0 = jnp.maximum(dinv * (a_ref[0] + h_ref[0]) + b_ref[0:1, 0:HALF], 0.0)
        t1 = jnp.maximum(dinv * (a_ref[1] + h_ref[1]) + b_ref[0:1, HALF:D], 0.0)
        g = (jnp.dot(t0, w_ref[0:HALF, :], preferred_element_type=jnp.float32,
                     precision=lax.Precision.HIGHEST)
             + jnp.dot(t1, w_ref[HALF:D, :], preferred_element_type=jnp.float32,
                       precision=lax.Precision.HIGHEST))
        g_ref[...] = g * dinv

    return pl.pallas_call(
        body,
        grid=(N // _BLK,),
        in_specs=[
            pl.BlockSpec((NC, _BLK, HALF), lambda i: (0, i, 0)),
            pl.BlockSpec((NC, _BLK, HALF), lambda i: (0, i, 0)),
            pl.BlockSpec((_BLK, 1), lambda i: (i, 0)),
            pl.BlockSpec((1, D), lambda i: (0, 0)),
            pl.BlockSpec((D, 16), lambda i: (0, 0)),
        ],
        out_specs=pl.BlockSpec((_BLK, 16), lambda i: (i, 0)),
        out_shape=jax.ShapeDtypeStruct((N, 16), jnp.float32),
    )(acc1, H, dinv, b1r, W2p)


def _tc3(acc2, g, dinv, b2p):
    def body(a_ref, g_ref, dinv_ref, b_ref, o_ref):
        o = dinv_ref[...] * (a_ref[0] + a_ref[1] + g_ref[...]) + b_ref[...]
        o_ref[...] = o[:, 0:3]

    return pl.pallas_call(
        body,
        grid=(1,),
        in_specs=[
            pl.BlockSpec((NC, N, 16), lambda i: (0, 0, 0)),
            pl.BlockSpec((N, 16), lambda i: (0, 0)),
            pl.BlockSpec((N, 1), lambda i: (0, 0)),
            pl.BlockSpec((1, 16), lambda i: (0, 0)),
        ],
        out_specs=pl.BlockSpec((N, 3), lambda i: (0, 0)),
        out_shape=jax.ShapeDtypeStruct((N, 3), jnp.float32),
    )(acc2, g, dinv, b2p)


def kernel(x, edge_index, W1, b1, W2, b2):
    src1d, dst1d = _split_edges(edge_index.astype(jnp.int32))
    src2d = src1d.reshape(NCH, 128)
    dst2d = dst1d.reshape(NCH, 128)
    src40 = src1d.reshape(E // 40, 40)
    dst40 = dst1d.reshape(E // 40, 40)

    zeros16 = jnp.zeros((RPT, 16), jnp.float32)
    zeros128 = jnp.zeros((RPT, HALF), jnp.float32)
    ones16 = jnp.ones((128, 16), jnp.float32)
    W2p = jnp.pad(W2, ((0, 0), (0, 16 - W2.shape[1])))
    b1r = b1.reshape(1, D)
    b2p = jnp.pad(b2, (0, 16 - b2.shape[0])).reshape(1, 16)

    degp = _deg_sc(dst2d, ones16, zeros16)
    H, dinv = _tc1(x, W1, degp)
    acc1 = _agg1_sc(H, src40, dst40, zeros128)
    g = _tc2(acc1, H, dinv, b1r, W2p)
    acc2 = _agg2_sc(g, src2d, dst2d, zeros16)
    return _tc3(acc2, g, dinv, b2p)
